# baseline (speedup 1.0000x reference)
import functools

import jax
import jax.numpy as jnp
from jax import lax
from jax.experimental import pallas as pl
from jax.experimental.pallas import tpu as pltpu
from jax.experimental.pallas import tpu_sc as plsc

N = 10000
E = 320000
D_IN = 128
EMB = 128
HID = 128
OUT = 64
L = 3

NC = 2
NS = 16
NW = NC * NS
EPW = E // NW
CH = 128
NCHUNK = 80
EPW_PAD = CH * NCHUNK
N_ACC = N + 8
RPB = 624
TAIL = N - NS * RPB

ROWS_B = 1000
GRID = N // ROWS_B


def _ln_rows(t, s, b):
    mu = jnp.mean(t, axis=-1, keepdims=True)
    var = jnp.mean((t - mu) ** 2, axis=-1, keepdims=True)
    return (t - mu) * lax.rsqrt(var + 1e-5) * s + b


def _in_body(x_ref, w_ref, b_ref, o_ref):
    o_ref[...] = jax.nn.relu(
        jnp.dot(x_ref[...], w_ref[...], preferred_element_type=jnp.float32)
        + b_ref[...]
    )


def _msg_body(h_ref, deg_ref, w_ref, b_ref, p_ref):
    inv = lax.rsqrt(jnp.maximum(deg_ref[...] + 1.0, 1.0))
    m = jax.nn.relu(
        jnp.dot(h_ref[...], w_ref[...], preferred_element_type=jnp.float32)
        + b_ref[...]
    )
    p_ref[...] = m * inv


def _upd_body(q0_ref, q1_ref, p_ref, h_ref, deg_ref, w_ref, b_ref, s_ref, lb_ref, o_ref):
    inv = lax.rsqrt(jnp.maximum(deg_ref[...] + 1.0, 1.0))
    agg = (q0_ref[...] + q1_ref[...] + p_ref[...]) * inv
    t = jnp.dot(agg, w_ref[...], preferred_element_type=jnp.float32) + b_ref[...]
    o_ref[...] = _ln_rows(t, s_ref[...], lb_ref[...]) + h_ref[...]


def _out_body(h_ref, s_ref, lb_ref, w_ref, b_ref, o_ref):
    t = _ln_rows(h_ref[...], s_ref[...], lb_ref[...])
    o_ref[...] = (
        jnp.dot(t, w_ref[...], preferred_element_type=jnp.float32) + b_ref[...]
    )


_in_call = pl.pallas_call(
    _in_body,
    grid=(GRID,),
    in_specs=[
        pl.BlockSpec((ROWS_B, D_IN), lambda i: (i, 0)),
        pl.BlockSpec((D_IN, EMB), lambda i: (0, 0)),
        pl.BlockSpec((1, EMB), lambda i: (0, 0)),
    ],
    out_specs=pl.BlockSpec((ROWS_B, EMB), lambda i: (i, 0)),
    out_shape=jax.ShapeDtypeStruct((N, EMB), jnp.float32),
)

_msg_call = pl.pallas_call(
    _msg_body,
    grid=(GRID,),
    in_specs=[
        pl.BlockSpec((ROWS_B, EMB), lambda i: (i, 0)),
        pl.BlockSpec((ROWS_B, 1), lambda i: (i, 0)),
        pl.BlockSpec((EMB, HID), lambda i: (0, 0)),
        pl.BlockSpec((1, HID), lambda i: (0, 0)),
    ],
    out_specs=pl.BlockSpec((ROWS_B, HID), lambda i: (i, 0)),
    out_shape=jax.ShapeDtypeStruct((N, HID), jnp.float32),
)

_upd_call = pl.pallas_call(
    _upd_body,
    grid=(GRID,),
    in_specs=[
        pl.BlockSpec((ROWS_B, HID), lambda i: (i, 0)),
        pl.BlockSpec((ROWS_B, HID), lambda i: (i, 0)),
        pl.BlockSpec((ROWS_B, HID), lambda i: (i, 0)),
        pl.BlockSpec((ROWS_B, EMB), lambda i: (i, 0)),
        pl.BlockSpec((ROWS_B, 1), lambda i: (i, 0)),
        pl.BlockSpec((HID, EMB), lambda i: (0, 0)),
        pl.BlockSpec((1, EMB), lambda i: (0, 0)),
        pl.BlockSpec((1, EMB), lambda i: (0, 0)),
        pl.BlockSpec((1, EMB), lambda i: (0, 0)),
    ],
    out_specs=pl.BlockSpec((ROWS_B, EMB), lambda i: (i, 0)),
    out_shape=jax.ShapeDtypeStruct((N, EMB), jnp.float32),
)

_out_call = pl.pallas_call(
    _out_body,
    grid=(GRID,),
    in_specs=[
        pl.BlockSpec((ROWS_B, EMB), lambda i: (i, 0)),
        pl.BlockSpec((1, EMB), lambda i: (0, 0)),
        pl.BlockSpec((1, EMB), lambda i: (0, 0)),
        pl.BlockSpec((EMB, OUT), lambda i: (0, 0)),
        pl.BlockSpec((1, OUT), lambda i: (0, 0)),
    ],
    out_specs=pl.BlockSpec((ROWS_B, OUT), lambda i: (i, 0)),
    out_shape=jax.ShapeDtypeStruct((N, OUT), jnp.float32),
)


def _sc_body(p_hbm, src_hbm, dst_hbm, zeros_hbm, out_hbm,
             src_v, dst_v, rows_v, acc, isem, gsem, ssem):
    c = lax.axis_index("c")
    s = lax.axis_index("s")
    wid = c * NS + s
    pltpu.sync_copy(zeros_hbm.at[pl.ds(0, RPB)], acc.at[pl.ds(s * RPB, RPB)])

    @pl.when(s == 0)
    def _zero_tail():
        pltpu.sync_copy(zeros_hbm.at[pl.ds(0, TAIL)],
                        acc.at[pl.ds(NS * RPB, TAIL)])

    plsc.subcore_barrier()

    def load_idx(ch, slot):
        pltpu.async_copy(src_hbm.at[wid, ch], src_v.at[slot], isem.at[slot])
        pltpu.async_copy(dst_hbm.at[wid, ch], dst_v.at[slot], isem.at[slot])

    def wait_idx(slot):
        pltpu.make_async_copy(src_hbm.at[wid, 0], src_v.at[slot],
                              isem.at[slot]).wait()
        pltpu.make_async_copy(dst_hbm.at[wid, 0], dst_v.at[slot],
                              isem.at[slot]).wait()

    load_idx(0, 0)
    load_idx(1, 1)
    wait_idx(0)
    pltpu.async_copy(p_hbm.at[src_v.at[0]], rows_v.at[0], gsem.at[0])

    def chunk(ci, carry):
        b = lax.rem(ci, 2)
        nb = 1 - b
        s3 = lax.rem(ci, 3)
        pltpu.make_async_copy(p_hbm.at[src_v.at[s3]], rows_v.at[b],
                              gsem.at[b]).wait()

        @pl.when(ci >= 1)
        def _wait_prev_scatter():
            pltpu.make_async_copy(rows_v.at[nb], acc.at[dst_v.at[s3]],
                                  ssem.at[nb]).wait()

        @pl.when(ci + 2 < NCHUNK)
        def _load_idx_ahead():
            load_idx(ci + 2, lax.rem(ci + 2, 3))

        @pl.when(ci + 1 < NCHUNK)
        def _next_gather():
            ns3 = lax.rem(ci + 1, 3)
            wait_idx(ns3)
            pltpu.async_copy(p_hbm.at[src_v.at[ns3]], rows_v.at[nb],
                             gsem.at[nb])

        pltpu.async_copy(rows_v.at[b], acc.at[dst_v.at[s3]], ssem.at[b],
                         add=True)
        return carry

    lax.fori_loop(0, NCHUNK, chunk, 0)
    last = (NCHUNK - 1) % 2
    pltpu.make_async_copy(rows_v.at[last],
                          acc.at[dst_v.at[(NCHUNK - 1) % 3]],
                          ssem.at[last]).wait()
    plsc.subcore_barrier()
    pltpu.sync_copy(acc.at[pl.ds(s * RPB, RPB)],
                    out_hbm.at[c].at[pl.ds(s * RPB, RPB)])

    @pl.when(s == 0)
    def _drain_tail():
        pltpu.sync_copy(acc.at[pl.ds(NS * RPB, TAIL)],
                        out_hbm.at[c].at[pl.ds(NS * RPB, TAIL)])


@functools.lru_cache(maxsize=None)
def _make_sc_call():
    return functools.partial(
        pl.kernel,
        out_type=jax.ShapeDtypeStruct((NC, N, EMB), jnp.float32),
        mesh=plsc.VectorSubcoreMesh(core_axis_name="c", subcore_axis_name="s",
                                    num_cores=NC, num_subcores=NS),
        scratch_types=[
            pltpu.VMEM((3, CH), jnp.int32),
            pltpu.VMEM((3, CH), jnp.int32),
            pltpu.VMEM((2, CH, EMB), jnp.float32),
            pltpu.VMEM_SHARED((N_ACC, EMB), jnp.float32),
            pltpu.SemaphoreType.DMA((3,)),
            pltpu.SemaphoreType.DMA((2,)),
            pltpu.SemaphoreType.DMA((2,)),
        ],
    )(_sc_body)


def kernel(x, degree, W_in, b_in, W1, b1, W2, b2, ln_s, ln_b,
           out_ln_s, out_ln_b, W_out, b_out, edge_index):
    deg = degree.reshape(N, 1)
    pad = EPW_PAD - EPW
    src = jnp.concatenate(
        [edge_index[0].reshape(NW, EPW),
         jnp.zeros((NW, pad), jnp.int32)], axis=1).reshape(NW, NCHUNK, CH)
    dst = jnp.concatenate(
        [edge_index[1].reshape(NW, EPW),
         jnp.full((NW, pad), N, jnp.int32)], axis=1).reshape(NW, NCHUNK, CH)
    zeros = jnp.zeros((RPB, EMB), jnp.float32)

    h = _in_call(x, W_in, b_in.reshape(1, EMB))
    for l in range(L):
        p = _msg_call(h, deg, W1[l], b1[l].reshape(1, HID))
        q = _make_sc_call()(p, src, dst, zeros)
        h = _upd_call(q[0], q[1], p, h, deg, W2[l], b2[l].reshape(1, EMB),
                      ln_s[l].reshape(1, EMB), ln_b[l].reshape(1, EMB))
    return _out_call(h, out_ln_s.reshape(1, EMB), out_ln_b.reshape(1, EMB),
                     W_out, b_out.reshape(1, OUT))


# speedup vs baseline: 1.0005x; 1.0005x over previous
import functools

import jax
import jax.numpy as jnp
from jax import lax
from jax.experimental import pallas as pl
from jax.experimental.pallas import tpu as pltpu
from jax.experimental.pallas import tpu_sc as plsc

N = 10000
E = 320000
D_IN = 128
EMB = 128
HID = 128
OUT = 64
L = 3

NC = 2
NS = 16
NW = NC * NS
EPW = E // NW
CH = 128
NCHUNK = 80
EPW_PAD = CH * NCHUNK
N_ACC = N + 16
RPB = 624
TAIL = N - NS * RPB

ROWS_B = 1000
GRID = N // ROWS_B


def _ln_rows(t, s, b):
    mu = jnp.mean(t, axis=-1, keepdims=True)
    var = jnp.mean((t - mu) ** 2, axis=-1, keepdims=True)
    return (t - mu) * lax.rsqrt(var + 1e-5) * s + b


def _in_body(x_ref, w_ref, b_ref, o_ref):
    o_ref[...] = jax.nn.relu(
        jnp.dot(x_ref[...], w_ref[...], preferred_element_type=jnp.float32)
        + b_ref[...]
    )


def _msg_body(h_ref, deg_ref, w_ref, b_ref, p_ref):
    inv = lax.rsqrt(jnp.maximum(deg_ref[...] + 1.0, 1.0))
    m = jax.nn.relu(
        jnp.dot(h_ref[...], w_ref[...], preferred_element_type=jnp.float32)
        + b_ref[...]
    )
    p_ref[...] = m * inv


def _upd_body(q0_ref, q1_ref, p_ref, h_ref, deg_ref, w_ref, b_ref, s_ref, lb_ref, o_ref):
    inv = lax.rsqrt(jnp.maximum(deg_ref[...] + 1.0, 1.0))
    agg = (q0_ref[...] + q1_ref[...] + p_ref[...]) * inv
    t = jnp.dot(agg, w_ref[...], preferred_element_type=jnp.float32) + b_ref[...]
    o_ref[...] = _ln_rows(t, s_ref[...], lb_ref[...]) + h_ref[...]


def _out_body(h_ref, s_ref, lb_ref, w_ref, b_ref, o_ref):
    t = _ln_rows(h_ref[...], s_ref[...], lb_ref[...])
    o_ref[...] = (
        jnp.dot(t, w_ref[...], preferred_element_type=jnp.float32) + b_ref[...]
    )


_in_call = pl.pallas_call(
    _in_body,
    grid=(GRID,),
    in_specs=[
        pl.BlockSpec((ROWS_B, D_IN), lambda i: (i, 0)),
        pl.BlockSpec((D_IN, EMB), lambda i: (0, 0)),
        pl.BlockSpec((1, EMB), lambda i: (0, 0)),
    ],
    out_specs=pl.BlockSpec((ROWS_B, EMB), lambda i: (i, 0)),
    out_shape=jax.ShapeDtypeStruct((N, EMB), jnp.float32),
)

_msg_call = pl.pallas_call(
    _msg_body,
    grid=(GRID,),
    in_specs=[
        pl.BlockSpec((ROWS_B, EMB), lambda i: (i, 0)),
        pl.BlockSpec((ROWS_B, 1), lambda i: (i, 0)),
        pl.BlockSpec((EMB, HID), lambda i: (0, 0)),
        pl.BlockSpec((1, HID), lambda i: (0, 0)),
    ],
    out_specs=pl.BlockSpec((ROWS_B, HID), lambda i: (i, 0)),
    out_shape=jax.ShapeDtypeStruct((N, HID), jnp.float32),
)

_upd_call = pl.pallas_call(
    _upd_body,
    grid=(GRID,),
    in_specs=[
        pl.BlockSpec((ROWS_B, HID), lambda i: (i, 0)),
        pl.BlockSpec((ROWS_B, HID), lambda i: (i, 0)),
        pl.BlockSpec((ROWS_B, HID), lambda i: (i, 0)),
        pl.BlockSpec((ROWS_B, EMB), lambda i: (i, 0)),
        pl.BlockSpec((ROWS_B, 1), lambda i: (i, 0)),
        pl.BlockSpec((HID, EMB), lambda i: (0, 0)),
        pl.BlockSpec((1, EMB), lambda i: (0, 0)),
        pl.BlockSpec((1, EMB), lambda i: (0, 0)),
        pl.BlockSpec((1, EMB), lambda i: (0, 0)),
    ],
    out_specs=pl.BlockSpec((ROWS_B, EMB), lambda i: (i, 0)),
    out_shape=jax.ShapeDtypeStruct((N, EMB), jnp.float32),
)

_out_call = pl.pallas_call(
    _out_body,
    grid=(GRID,),
    in_specs=[
        pl.BlockSpec((ROWS_B, EMB), lambda i: (i, 0)),
        pl.BlockSpec((1, EMB), lambda i: (0, 0)),
        pl.BlockSpec((1, EMB), lambda i: (0, 0)),
        pl.BlockSpec((EMB, OUT), lambda i: (0, 0)),
        pl.BlockSpec((1, OUT), lambda i: (0, 0)),
    ],
    out_specs=pl.BlockSpec((ROWS_B, OUT), lambda i: (i, 0)),
    out_shape=jax.ShapeDtypeStruct((N, OUT), jnp.float32),
)


def _sc_body(p_hbm, src_hbm, dst_hbm, zeros_hbm, out_hbm,
             src_v, dst_v, rows_v, acc, isem, gsem, ssem):
    c = lax.axis_index("c")
    s = lax.axis_index("s")
    wid = c * NS + s
    pltpu.sync_copy(zeros_hbm.at[pl.ds(0, RPB)], acc.at[pl.ds(s * RPB, RPB)])

    @pl.when(s == 0)
    def _zero_tail():
        pltpu.sync_copy(zeros_hbm.at[pl.ds(0, TAIL)],
                        acc.at[pl.ds(NS * RPB, TAIL)])

    plsc.subcore_barrier()

    def load_idx(ch, slot):
        pltpu.async_copy(src_hbm.at[wid, ch], src_v.at[slot], isem.at[slot])
        pltpu.async_copy(dst_hbm.at[wid, ch], dst_v.at[slot], isem.at[slot])

    def wait_idx(slot):
        pltpu.make_async_copy(src_hbm.at[wid, 0], src_v.at[slot],
                              isem.at[slot]).wait()
        pltpu.make_async_copy(dst_hbm.at[wid, 0], dst_v.at[slot],
                              isem.at[slot]).wait()

    load_idx(0, 0)
    load_idx(1, 1)
    wait_idx(0)
    pltpu.async_copy(p_hbm.at[src_v.at[0]], rows_v.at[0], gsem.at[0])

    def chunk(ci, carry):
        b = lax.rem(ci, 2)
        nb = 1 - b
        s3 = lax.rem(ci, 3)
        pltpu.make_async_copy(p_hbm.at[src_v.at[s3]], rows_v.at[b],
                              gsem.at[b]).wait()

        @pl.when(ci >= 1)
        def _wait_prev_scatter():
            pltpu.make_async_copy(rows_v.at[nb], acc.at[dst_v.at[s3]],
                                  ssem.at[nb]).wait()

        @pl.when(ci + 2 < NCHUNK)
        def _load_idx_ahead():
            load_idx(ci + 2, lax.rem(ci + 2, 3))

        @pl.when(ci + 1 < NCHUNK)
        def _next_gather():
            ns3 = lax.rem(ci + 1, 3)
            wait_idx(ns3)
            pltpu.async_copy(p_hbm.at[src_v.at[ns3]], rows_v.at[nb],
                             gsem.at[nb])

        pltpu.async_copy(rows_v.at[b], acc.at[dst_v.at[s3]], ssem.at[b],
                         add=True)
        return carry

    lax.fori_loop(0, NCHUNK, chunk, 0)
    last = (NCHUNK - 1) % 2
    pltpu.make_async_copy(rows_v.at[last],
                          acc.at[dst_v.at[(NCHUNK - 1) % 3]],
                          ssem.at[last]).wait()
    plsc.subcore_barrier()
    pltpu.sync_copy(acc.at[pl.ds(s * RPB, RPB)],
                    out_hbm.at[c].at[pl.ds(s * RPB, RPB)])

    @pl.when(s == 0)
    def _drain_tail():
        pltpu.sync_copy(acc.at[pl.ds(NS * RPB, TAIL)],
                        out_hbm.at[c].at[pl.ds(NS * RPB, TAIL)])


@functools.lru_cache(maxsize=None)
def _make_sc_call():
    return functools.partial(
        pl.kernel,
        out_type=jax.ShapeDtypeStruct((NC, N, EMB), jnp.float32),
        mesh=plsc.VectorSubcoreMesh(core_axis_name="c", subcore_axis_name="s",
                                    num_cores=NC, num_subcores=NS),
        scratch_types=[
            pltpu.VMEM((3, CH), jnp.int32),
            pltpu.VMEM((3, CH), jnp.int32),
            pltpu.VMEM((2, CH, EMB), jnp.float32),
            pltpu.VMEM_SHARED((N_ACC, EMB), jnp.float32),
            pltpu.SemaphoreType.DMA((3,)),
            pltpu.SemaphoreType.DMA((2,)),
            pltpu.SemaphoreType.DMA((2,)),
        ],
    )(_sc_body)


def kernel(x, degree, W_in, b_in, W1, b1, W2, b2, ln_s, ln_b,
           out_ln_s, out_ln_b, W_out, b_out, edge_index):
    deg = degree.reshape(N, 1)
    pad = EPW_PAD - EPW
    src = jnp.concatenate(
        [edge_index[0].reshape(NW, EPW),
         jnp.zeros((NW, pad), jnp.int32)], axis=1).reshape(NW, NCHUNK, CH)
    junk = N + (jnp.arange(NW, dtype=jnp.int32) % NS)
    dst = jnp.concatenate(
        [edge_index[1].reshape(NW, EPW),
         jnp.broadcast_to(junk[:, None], (NW, pad))],
        axis=1).reshape(NW, NCHUNK, CH)
    zeros = jnp.zeros((RPB, EMB), jnp.float32)

    h = _in_call(x, W_in, b_in.reshape(1, EMB))
    for l in range(L):
        p = _msg_call(h, deg, W1[l], b1[l].reshape(1, HID))
        q = _make_sc_call()(p, src, dst, zeros)
        h = _upd_call(q[0], q[1], p, h, deg, W2[l], b2[l].reshape(1, EMB),
                      ln_s[l].reshape(1, EMB), ln_b[l].reshape(1, EMB))
    return _out_call(h, out_ln_s.reshape(1, EMB), out_ln_b.reshape(1, EMB),
                     W_out, b_out.reshape(1, OUT))


# speedup vs baseline: 1.0021x; 1.0017x over previous
import functools

import jax
import jax.numpy as jnp
from jax import lax
from jax.experimental import pallas as pl
from jax.experimental.pallas import tpu as pltpu
from jax.experimental.pallas import tpu_sc as plsc

N = 10000
E = 320000
D_IN = 128
EMB = 128
HID = 128
OUT = 64
L = 3

NC = 2
NS = 16
NW = NC * NS
EPW = E // NW
CH = 128
NCHUNK = 80
EPW_PAD = CH * NCHUNK
N_ACC = N + 16
RPB = 624
TAIL = N - NS * RPB

ROWS_B = 1000
GRID = N // ROWS_B


def _ln_rows(t, s, b):
    mu = jnp.mean(t, axis=-1, keepdims=True)
    var = jnp.mean((t - mu) ** 2, axis=-1, keepdims=True)
    return (t - mu) * lax.rsqrt(var + 1e-5) * s + b


def _in_body(x_ref, w_ref, b_ref, o_ref):
    o_ref[...] = jax.nn.relu(
        jnp.dot(x_ref[...], w_ref[...], preferred_element_type=jnp.float32)
        + b_ref[...]
    )


def _msg_body(h_ref, deg_ref, w_ref, b_ref, p_ref):
    inv = lax.rsqrt(jnp.maximum(deg_ref[...] + 1.0, 1.0))
    m = jax.nn.relu(
        jnp.dot(h_ref[...], w_ref[...], preferred_element_type=jnp.float32)
        + b_ref[...]
    )
    p_ref[...] = m * inv


def _upd_body(q0_ref, q1_ref, p_ref, h_ref, deg_ref, w_ref, b_ref, s_ref, lb_ref, o_ref):
    inv = lax.rsqrt(jnp.maximum(deg_ref[...] + 1.0, 1.0))
    agg = (q0_ref[...] + q1_ref[...] + p_ref[...]) * inv
    t = jnp.dot(agg, w_ref[...], preferred_element_type=jnp.float32) + b_ref[...]
    o_ref[...] = _ln_rows(t, s_ref[...], lb_ref[...]) + h_ref[...]


def _out_body(h_ref, s_ref, lb_ref, w_ref, b_ref, o_ref):
    t = _ln_rows(h_ref[...], s_ref[...], lb_ref[...])
    o_ref[...] = (
        jnp.dot(t, w_ref[...], preferred_element_type=jnp.float32) + b_ref[...]
    )


_in_call = pl.pallas_call(
    _in_body,
    grid=(GRID,),
    in_specs=[
        pl.BlockSpec((ROWS_B, D_IN), lambda i: (i, 0)),
        pl.BlockSpec((D_IN, EMB), lambda i: (0, 0)),
        pl.BlockSpec((1, EMB), lambda i: (0, 0)),
    ],
    out_specs=pl.BlockSpec((ROWS_B, EMB), lambda i: (i, 0)),
    out_shape=jax.ShapeDtypeStruct((N, EMB), jnp.float32),
)

_msg_call = pl.pallas_call(
    _msg_body,
    grid=(GRID,),
    in_specs=[
        pl.BlockSpec((ROWS_B, EMB), lambda i: (i, 0)),
        pl.BlockSpec((ROWS_B, 1), lambda i: (i, 0)),
        pl.BlockSpec((EMB, HID), lambda i: (0, 0)),
        pl.BlockSpec((1, HID), lambda i: (0, 0)),
    ],
    out_specs=pl.BlockSpec((ROWS_B, HID), lambda i: (i, 0)),
    out_shape=jax.ShapeDtypeStruct((N, HID), jnp.float32),
)

_upd_call = pl.pallas_call(
    _upd_body,
    grid=(GRID,),
    in_specs=[
        pl.BlockSpec((ROWS_B, HID), lambda i: (i, 0)),
        pl.BlockSpec((ROWS_B, HID), lambda i: (i, 0)),
        pl.BlockSpec((ROWS_B, HID), lambda i: (i, 0)),
        pl.BlockSpec((ROWS_B, EMB), lambda i: (i, 0)),
        pl.BlockSpec((ROWS_B, 1), lambda i: (i, 0)),
        pl.BlockSpec((HID, EMB), lambda i: (0, 0)),
        pl.BlockSpec((1, EMB), lambda i: (0, 0)),
        pl.BlockSpec((1, EMB), lambda i: (0, 0)),
        pl.BlockSpec((1, EMB), lambda i: (0, 0)),
    ],
    out_specs=pl.BlockSpec((ROWS_B, EMB), lambda i: (i, 0)),
    out_shape=jax.ShapeDtypeStruct((N, EMB), jnp.float32),
)

_out_call = pl.pallas_call(
    _out_body,
    grid=(GRID,),
    in_specs=[
        pl.BlockSpec((ROWS_B, EMB), lambda i: (i, 0)),
        pl.BlockSpec((1, EMB), lambda i: (0, 0)),
        pl.BlockSpec((1, EMB), lambda i: (0, 0)),
        pl.BlockSpec((EMB, OUT), lambda i: (0, 0)),
        pl.BlockSpec((1, OUT), lambda i: (0, 0)),
    ],
    out_specs=pl.BlockSpec((ROWS_B, OUT), lambda i: (i, 0)),
    out_shape=jax.ShapeDtypeStruct((N, OUT), jnp.float32),
)


def _sc_body(p_hbm, src_hbm, dst_hbm, zeros_hbm, out_hbm,
             src_v, dst_v, rows_v, acc, isem, gsem, ssem):
    c = lax.axis_index("c")
    s = lax.axis_index("s")
    wid = c * NS + s
    pltpu.sync_copy(zeros_hbm.at[pl.ds(0, RPB)], acc.at[pl.ds(s * RPB, RPB)])

    @pl.when(s == 0)
    def _zero_tail():
        pltpu.sync_copy(zeros_hbm.at[pl.ds(0, TAIL)],
                        acc.at[pl.ds(NS * RPB, TAIL)])

    plsc.subcore_barrier()

    def load_idx(ch, slot):
        pltpu.async_copy(src_hbm.at[wid, ch], src_v.at[slot], isem.at[slot])
        pltpu.async_copy(dst_hbm.at[wid, ch], dst_v.at[slot], isem.at[slot])

    def wait_idx(slot):
        pltpu.make_async_copy(src_hbm.at[wid, 0], src_v.at[slot],
                              isem.at[slot]).wait()
        pltpu.make_async_copy(dst_hbm.at[wid, 0], dst_v.at[slot],
                              isem.at[slot]).wait()

    def issue_gather(slot, b):
        pltpu.async_copy(p_hbm.at[src_v.at[slot]], rows_v.at[b], gsem.at[b])

    def wait_gather(slot, b):
        pltpu.make_async_copy(p_hbm.at[src_v.at[slot]], rows_v.at[b],
                              gsem.at[b]).wait()

    def issue_scatter(slot, b):
        pltpu.async_copy(rows_v.at[b], acc.at[dst_v.at[slot]], ssem.at[b],
                         add=True)

    def wait_scatter(slot, b):
        pltpu.make_async_copy(rows_v.at[b], acc.at[dst_v.at[slot]],
                              ssem.at[b]).wait()

    def step(cc, j, do_swait, do_load, do_gather):
        b = j % 2
        nb = 1 - b
        wait_gather(j, b)
        if do_swait:
            wait_scatter((j - 1) % 4, nb)
        if do_load:
            load_idx(cc + 3, (j + 3) % 4)
        if do_gather:
            wait_idx((j + 1) % 4)
            issue_gather((j + 1) % 4, nb)
        issue_scatter(j, b)

    for sl in range(3):
        load_idx(sl, sl)
    wait_idx(0)
    issue_gather(0, 0)
    for j in range(4):
        step(j, j, do_swait=(j > 0), do_load=True, do_gather=True)

    def macro(i, carry):
        c0 = 4 * i
        for j in range(4):
            step(c0 + j, j, True, True, True)
        return carry

    lax.fori_loop(1, NCHUNK // 4 - 1, macro, 0)
    for j in range(4):
        step(NCHUNK - 4 + j, j, do_swait=True, do_load=(j == 0),
             do_gather=(j < 3))
    wait_scatter(3, 1)
    plsc.subcore_barrier()
    pltpu.sync_copy(acc.at[pl.ds(s * RPB, RPB)],
                    out_hbm.at[c].at[pl.ds(s * RPB, RPB)])

    @pl.when(s == 0)
    def _drain_tail():
        pltpu.sync_copy(acc.at[pl.ds(NS * RPB, TAIL)],
                        out_hbm.at[c].at[pl.ds(NS * RPB, TAIL)])


@functools.lru_cache(maxsize=None)
def _make_sc_call():
    return functools.partial(
        pl.kernel,
        out_type=jax.ShapeDtypeStruct((NC, N, EMB), jnp.float32),
        mesh=plsc.VectorSubcoreMesh(core_axis_name="c", subcore_axis_name="s",
                                    num_cores=NC, num_subcores=NS),
        scratch_types=[
            pltpu.VMEM((4, CH), jnp.int32),
            pltpu.VMEM((4, CH), jnp.int32),
            pltpu.VMEM((2, CH, EMB), jnp.float32),
            pltpu.VMEM_SHARED((N_ACC, EMB), jnp.float32),
            pltpu.SemaphoreType.DMA((4,)),
            pltpu.SemaphoreType.DMA((2,)),
            pltpu.SemaphoreType.DMA((2,)),
        ],
    )(_sc_body)


def kernel(x, degree, W_in, b_in, W1, b1, W2, b2, ln_s, ln_b,
           out_ln_s, out_ln_b, W_out, b_out, edge_index):
    deg = degree.reshape(N, 1)
    pad = EPW_PAD - EPW
    src = jnp.concatenate(
        [edge_index[0].reshape(NW, EPW),
         jnp.zeros((NW, pad), jnp.int32)], axis=1).reshape(NW, NCHUNK, CH)
    junk = N + (jnp.arange(NW, dtype=jnp.int32) % NS)
    dst = jnp.concatenate(
        [edge_index[1].reshape(NW, EPW),
         jnp.broadcast_to(junk[:, None], (NW, pad))],
        axis=1).reshape(NW, NCHUNK, CH)
    zeros = jnp.zeros((RPB, EMB), jnp.float32)

    h = _in_call(x, W_in, b_in.reshape(1, EMB))
    for l in range(L):
        p = _msg_call(h, deg, W1[l], b1[l].reshape(1, HID))
        q = _make_sc_call()(p, src, dst, zeros)
        h = _upd_call(q[0], q[1], p, h, deg, W2[l], b2[l].reshape(1, EMB),
                      ln_s[l].reshape(1, EMB), ln_b[l].reshape(1, EMB))
    return _out_call(h, out_ln_s.reshape(1, EMB), out_ln_b.reshape(1, EMB),
                     W_out, b_out.reshape(1, OUT))


# speedup vs baseline: 2.2168x; 2.2121x over previous
import functools

import jax
import jax.numpy as jnp
from jax import lax
from jax.experimental import pallas as pl
from jax.experimental.pallas import tpu as pltpu
from jax.experimental.pallas import tpu_sc as plsc

N = 10000
E = 320000
D_IN = 128
EMB = 128
HID = 128
OUT = 64
L = 3

NC = 2
NS = 16
NW = NC * NS
EPW = E // NW
CH = 80
NCHUNK = EPW // CH
SEG = 5
CPS = NCHUNK // SEG
RPB = 624
TAIL = N - NS * RPB

ROWS_B = 1000
GRID = N // ROWS_B


def _ln_rows(t, s, b):
    mu = jnp.mean(t, axis=-1, keepdims=True)
    var = jnp.mean((t - mu) ** 2, axis=-1, keepdims=True)
    return (t - mu) * lax.rsqrt(var + 1e-5) * s + b


def _in_body(x_ref, w_ref, b_ref, o_ref):
    o_ref[...] = jax.nn.relu(
        jnp.dot(x_ref[...], w_ref[...], preferred_element_type=jnp.float32)
        + b_ref[...]
    )


def _msg_body(h_ref, deg_ref, w_ref, b_ref, p_ref):
    inv = lax.rsqrt(jnp.maximum(deg_ref[...] + 1.0, 1.0))
    m = jax.nn.relu(
        jnp.dot(h_ref[...], w_ref[...], preferred_element_type=jnp.float32)
        + b_ref[...]
    )
    p_ref[...] = m * inv


def _upd_body(q0_ref, q1_ref, p_ref, h_ref, deg_ref, w_ref, b_ref, s_ref, lb_ref, o_ref):
    inv = lax.rsqrt(jnp.maximum(deg_ref[...] + 1.0, 1.0))
    agg = (q0_ref[...] + q1_ref[...] + p_ref[...]) * inv
    t = jnp.dot(agg, w_ref[...], preferred_element_type=jnp.float32) + b_ref[...]
    o_ref[...] = _ln_rows(t, s_ref[...], lb_ref[...]) + h_ref[...]


def _out_body(h_ref, s_ref, lb_ref, w_ref, b_ref, o_ref):
    t = _ln_rows(h_ref[...], s_ref[...], lb_ref[...])
    o_ref[...] = (
        jnp.dot(t, w_ref[...], preferred_element_type=jnp.float32) + b_ref[...]
    )


_in_call = pl.pallas_call(
    _in_body,
    grid=(GRID,),
    in_specs=[
        pl.BlockSpec((ROWS_B, D_IN), lambda i: (i, 0)),
        pl.BlockSpec((D_IN, EMB), lambda i: (0, 0)),
        pl.BlockSpec((1, EMB), lambda i: (0, 0)),
    ],
    out_specs=pl.BlockSpec((ROWS_B, EMB), lambda i: (i, 0)),
    out_shape=jax.ShapeDtypeStruct((N, EMB), jnp.float32),
)

_msg_call = pl.pallas_call(
    _msg_body,
    grid=(GRID,),
    in_specs=[
        pl.BlockSpec((ROWS_B, EMB), lambda i: (i, 0)),
        pl.BlockSpec((ROWS_B, 1), lambda i: (i, 0)),
        pl.BlockSpec((EMB, HID), lambda i: (0, 0)),
        pl.BlockSpec((1, HID), lambda i: (0, 0)),
    ],
    out_specs=pl.BlockSpec((ROWS_B, HID), lambda i: (i, 0)),
    out_shape=jax.ShapeDtypeStruct((N, HID), jnp.float32),
)

_upd_call = pl.pallas_call(
    _upd_body,
    grid=(GRID,),
    in_specs=[
        pl.BlockSpec((ROWS_B, HID), lambda i: (i, 0)),
        pl.BlockSpec((ROWS_B, HID), lambda i: (i, 0)),
        pl.BlockSpec((ROWS_B, HID), lambda i: (i, 0)),
        pl.BlockSpec((ROWS_B, EMB), lambda i: (i, 0)),
        pl.BlockSpec((ROWS_B, 1), lambda i: (i, 0)),
        pl.BlockSpec((HID, EMB), lambda i: (0, 0)),
        pl.BlockSpec((1, EMB), lambda i: (0, 0)),
        pl.BlockSpec((1, EMB), lambda i: (0, 0)),
        pl.BlockSpec((1, EMB), lambda i: (0, 0)),
    ],
    out_specs=pl.BlockSpec((ROWS_B, EMB), lambda i: (i, 0)),
    out_shape=jax.ShapeDtypeStruct((N, EMB), jnp.float32),
)

_out_call = pl.pallas_call(
    _out_body,
    grid=(GRID,),
    in_specs=[
        pl.BlockSpec((ROWS_B, EMB), lambda i: (i, 0)),
        pl.BlockSpec((1, EMB), lambda i: (0, 0)),
        pl.BlockSpec((1, EMB), lambda i: (0, 0)),
        pl.BlockSpec((EMB, OUT), lambda i: (0, 0)),
        pl.BlockSpec((1, OUT), lambda i: (0, 0)),
    ],
    out_specs=pl.BlockSpec((ROWS_B, OUT), lambda i: (i, 0)),
    out_shape=jax.ShapeDtypeStruct((N, OUT), jnp.float32),
)


def _sc_body(p_hbm, src_hbm, dst_hbm, zeros_hbm, out_hbm,
             src_a, dst_a, src_b, dst_b, rows0, rows1, acc,
             lsem, gsem, ssem):
    rows = (rows0, rows1)
    slabs = ((src_a, dst_a), (src_b, dst_b))
    c = lax.axis_index("c")
    s = lax.axis_index("s")
    wid = c * NS + s

    def load_slab(seg, t):
        pltpu.async_copy(src_hbm.at[wid, seg], slabs[t][0], lsem.at[t])
        pltpu.async_copy(dst_hbm.at[wid, seg], slabs[t][1], lsem.at[t])

    def wait_slab(t):
        pltpu.make_async_copy(src_hbm.at[wid, 0], slabs[t][0],
                              lsem.at[t]).wait()
        pltpu.make_async_copy(dst_hbm.at[wid, 0], slabs[t][1],
                              lsem.at[t]).wait()

    load_slab(0, 0)
    load_slab(1, 1)
    pltpu.sync_copy(zeros_hbm.at[pl.ds(0, RPB)], acc.at[pl.ds(s * RPB, RPB)])

    @pl.when(s == 0)
    def _zero_tail():
        pltpu.sync_copy(zeros_hbm.at[pl.ds(0, TAIL)],
                        acc.at[pl.ds(NS * RPB, TAIL)])

    plsc.subcore_barrier()

    def issue_gather(sv, cc, b):
        pltpu.async_copy(p_hbm.at[sv.at[cc]], rows[b], gsem.at[b])

    def wait_gather(sv, cc, b):
        pltpu.make_async_copy(p_hbm.at[sv.at[cc]], rows[b],
                              gsem.at[b]).wait()

    def issue_scatter(dv, cc, b):
        pltpu.async_copy(rows[b], acc.at[dv.at[cc]], ssem.at[b], add=True)

    def wait_scatter(dv, cc, b):
        pltpu.make_async_copy(rows[b], acc.at[dv.at[cc]],
                              ssem.at[b]).wait()

    def step(sv, dv, cc, b, do_swait, do_gather):
        nb = 1 - b
        wait_gather(sv, cc, b)
        if do_swait:
            wait_scatter(dv, cc - 1, nb)
        if do_gather:
            issue_gather(sv, cc + 1, nb)
        issue_scatter(dv, cc, b)

    for seg in range(SEG):
        t = seg % 2
        sv, dv = slabs[t]
        wait_slab(t)
        issue_gather(sv, 0, 0)
        step(sv, dv, 0, 0, do_swait=False, do_gather=True)

        def pair(j, carry, sv=sv, dv=dv):
            cb = 2 * j + 1
            step(sv, dv, cb, 1, True, True)
            step(sv, dv, cb + 1, 0, True, True)
            return carry

        lax.fori_loop(0, (CPS - 3) // 2, pair, 0)
        step(sv, dv, CPS - 2, 1, True, True)
        step(sv, dv, CPS - 1, 0, True, False)
        wait_scatter(dv, CPS - 1, 0)
        if seg + 2 < SEG:
            load_slab(seg + 2, t)

    plsc.subcore_barrier()
    pltpu.sync_copy(acc.at[pl.ds(s * RPB, RPB)],
                    out_hbm.at[c].at[pl.ds(s * RPB, RPB)])

    @pl.when(s == 0)
    def _drain_tail():
        pltpu.sync_copy(acc.at[pl.ds(NS * RPB, TAIL)],
                        out_hbm.at[c].at[pl.ds(NS * RPB, TAIL)])


@functools.lru_cache(maxsize=None)
def _make_sc_call():
    return functools.partial(
        pl.kernel,
        out_type=jax.ShapeDtypeStruct((NC, N, EMB), jnp.float32),
        mesh=plsc.VectorSubcoreMesh(core_axis_name="c", subcore_axis_name="s",
                                    num_cores=NC, num_subcores=NS),
        scratch_types=[
            pltpu.VMEM((CPS, CH), jnp.int32),
            pltpu.VMEM((CPS, CH), jnp.int32),
            pltpu.VMEM((CPS, CH), jnp.int32),
            pltpu.VMEM((CPS, CH), jnp.int32),
            pltpu.VMEM((CH, EMB), jnp.float32),
            pltpu.VMEM((CH, EMB), jnp.float32),
            pltpu.VMEM_SHARED((N, EMB), jnp.float32),
            pltpu.SemaphoreType.DMA((2,)),
            pltpu.SemaphoreType.DMA((2,)),
            pltpu.SemaphoreType.DMA((2,)),
        ],
    )(_sc_body)


def kernel(x, degree, W_in, b_in, W1, b1, W2, b2, ln_s, ln_b,
           out_ln_s, out_ln_b, W_out, b_out, edge_index):
    deg = degree.reshape(N, 1)
    src = edge_index[0].reshape(NW, SEG, CPS, CH)
    dst = edge_index[1].reshape(NW, SEG, CPS, CH)
    zeros = jnp.zeros((RPB, EMB), jnp.float32)

    h = _in_call(x, W_in, b_in.reshape(1, EMB))
    for l in range(L):
        p = _msg_call(h, deg, W1[l], b1[l].reshape(1, HID))
        q = _make_sc_call()(p, src, dst, zeros)
        h = _upd_call(q[0], q[1], p, h, deg, W2[l], b2[l].reshape(1, EMB),
                      ln_s[l].reshape(1, EMB), ln_b[l].reshape(1, EMB))
    return _out_call(h, out_ln_s.reshape(1, EMB), out_ln_b.reshape(1, EMB),
                     W_out, b_out.reshape(1, OUT))


# speedup vs baseline: 2.9299x; 1.3217x over previous
import functools

import jax
import jax.numpy as jnp
from jax import lax
from jax.experimental import pallas as pl
from jax.experimental.pallas import tpu as pltpu
from jax.experimental.pallas import tpu_sc as plsc

N = 10000
E = 320000
D_IN = 128
EMB = 128
HID = 128
OUT = 64
L = 3

NC = 2
NS = 16
NW = NC * NS
EPW = E // NW
CH = 80
NCHUNK = EPW // CH
SEG = 5
CPS = NCHUNK // SEG
RPB = 624
TAIL = N - NS * RPB

ROWS_B = 1000
GRID = N // ROWS_B


def _ln_rows(t, s, b):
    mu = jnp.mean(t, axis=-1, keepdims=True)
    var = jnp.mean((t - mu) ** 2, axis=-1, keepdims=True)
    return (t - mu) * lax.rsqrt(var + 1e-5) * s + b


def _in_body(x_ref, w_ref, b_ref, o_ref):
    o_ref[...] = jax.nn.relu(
        jnp.dot(x_ref[...], w_ref[...], preferred_element_type=jnp.float32)
        + b_ref[...]
    )


def _msg_body(h_ref, deg_ref, w_ref, b_ref, p_ref):
    inv = lax.rsqrt(jnp.maximum(deg_ref[...] + 1.0, 1.0))
    m = jax.nn.relu(
        jnp.dot(h_ref[...], w_ref[...], preferred_element_type=jnp.float32)
        + b_ref[...]
    )
    p_ref[...] = m * inv


def _upd_body(q0_ref, q1_ref, p_ref, h_ref, deg_ref, w_ref, b_ref, s_ref, lb_ref, o_ref):
    inv = lax.rsqrt(jnp.maximum(deg_ref[...] + 1.0, 1.0))
    agg = (q0_ref[...] + q1_ref[...] + p_ref[...]) * inv
    t = jnp.dot(agg, w_ref[...], preferred_element_type=jnp.float32) + b_ref[...]
    o_ref[...] = _ln_rows(t, s_ref[...], lb_ref[...]) + h_ref[...]


def _out_body(h_ref, s_ref, lb_ref, w_ref, b_ref, o_ref):
    t = _ln_rows(h_ref[...], s_ref[...], lb_ref[...])
    o_ref[...] = (
        jnp.dot(t, w_ref[...], preferred_element_type=jnp.float32) + b_ref[...]
    )


_in_call = pl.pallas_call(
    _in_body,
    grid=(GRID,),
    in_specs=[
        pl.BlockSpec((ROWS_B, D_IN), lambda i: (i, 0)),
        pl.BlockSpec((D_IN, EMB), lambda i: (0, 0)),
        pl.BlockSpec((1, EMB), lambda i: (0, 0)),
    ],
    out_specs=pl.BlockSpec((ROWS_B, EMB), lambda i: (i, 0)),
    out_shape=jax.ShapeDtypeStruct((N, EMB), jnp.float32),
)

_msg_call = pl.pallas_call(
    _msg_body,
    grid=(GRID,),
    in_specs=[
        pl.BlockSpec((ROWS_B, EMB), lambda i: (i, 0)),
        pl.BlockSpec((ROWS_B, 1), lambda i: (i, 0)),
        pl.BlockSpec((EMB, HID), lambda i: (0, 0)),
        pl.BlockSpec((1, HID), lambda i: (0, 0)),
    ],
    out_specs=pl.BlockSpec((ROWS_B, HID), lambda i: (i, 0)),
    out_shape=jax.ShapeDtypeStruct((N, HID), jnp.float32),
)

_upd_call = pl.pallas_call(
    _upd_body,
    grid=(GRID,),
    in_specs=[
        pl.BlockSpec((ROWS_B, HID), lambda i: (i, 0)),
        pl.BlockSpec((ROWS_B, HID), lambda i: (i, 0)),
        pl.BlockSpec((ROWS_B, HID), lambda i: (i, 0)),
        pl.BlockSpec((ROWS_B, EMB), lambda i: (i, 0)),
        pl.BlockSpec((ROWS_B, 1), lambda i: (i, 0)),
        pl.BlockSpec((HID, EMB), lambda i: (0, 0)),
        pl.BlockSpec((1, EMB), lambda i: (0, 0)),
        pl.BlockSpec((1, EMB), lambda i: (0, 0)),
        pl.BlockSpec((1, EMB), lambda i: (0, 0)),
    ],
    out_specs=pl.BlockSpec((ROWS_B, EMB), lambda i: (i, 0)),
    out_shape=jax.ShapeDtypeStruct((N, EMB), jnp.float32),
)

_out_call = pl.pallas_call(
    _out_body,
    grid=(GRID,),
    in_specs=[
        pl.BlockSpec((ROWS_B, EMB), lambda i: (i, 0)),
        pl.BlockSpec((1, EMB), lambda i: (0, 0)),
        pl.BlockSpec((1, EMB), lambda i: (0, 0)),
        pl.BlockSpec((EMB, OUT), lambda i: (0, 0)),
        pl.BlockSpec((1, OUT), lambda i: (0, 0)),
    ],
    out_specs=pl.BlockSpec((ROWS_B, OUT), lambda i: (i, 0)),
    out_shape=jax.ShapeDtypeStruct((N, OUT), jnp.float32),
)


def _sc_body(p_hbm, src_hbm, dst_hbm, zeros_hbm, out_hbm,
             src_a, dst_a, src_b, dst_b, rows0, rows1, rows2, acc,
             lsem, gsem, ssem):
    rows = (rows0, rows1, rows2)
    slabs = ((src_a, dst_a), (src_b, dst_b))
    c = lax.axis_index("c")
    s = lax.axis_index("s")
    wid = c * NS + s

    def load_slab(seg, t):
        pltpu.async_copy(src_hbm.at[wid, seg], slabs[t][0], lsem.at[t])
        pltpu.async_copy(dst_hbm.at[wid, seg], slabs[t][1], lsem.at[t])

    def wait_slab(t):
        pltpu.make_async_copy(src_hbm.at[wid, 0], slabs[t][0],
                              lsem.at[t]).wait()
        pltpu.make_async_copy(dst_hbm.at[wid, 0], slabs[t][1],
                              lsem.at[t]).wait()

    load_slab(0, 0)
    load_slab(1, 1)
    pltpu.sync_copy(zeros_hbm.at[pl.ds(0, RPB)], acc.at[pl.ds(s * RPB, RPB)])

    @pl.when(s == 0)
    def _zero_tail():
        pltpu.sync_copy(zeros_hbm.at[pl.ds(0, TAIL)],
                        acc.at[pl.ds(NS * RPB, TAIL)])

    plsc.subcore_barrier()

    def issue_gather(sv, cc, b):
        pltpu.async_copy(p_hbm.at[sv.at[cc]], rows[b], gsem.at[b])

    def wait_gather(sv, cc, b):
        pltpu.make_async_copy(p_hbm.at[sv.at[cc]], rows[b],
                              gsem.at[b]).wait()

    def issue_scatter(dv, cc, b):
        pltpu.async_copy(rows[b], acc.at[dv.at[cc]], ssem.at[b], add=True)

    def wait_scatter(dv, cc, b):
        pltpu.make_async_copy(rows[b], acc.at[dv.at[cc]],
                              ssem.at[b]).wait()

    def step(sv, dv, cc, b, do_swait, do_gather):
        wait_gather(sv, cc, b)
        if do_swait:
            wait_scatter(dv, cc - 1, (b + 2) % 3)
        if do_gather:
            issue_gather(sv, cc + 2, (b + 2) % 3)
        issue_scatter(dv, cc, b)

    for seg in range(SEG):
        t = seg % 2
        sv, dv = slabs[t]
        wait_slab(t)
        issue_gather(sv, 0, 0)
        issue_gather(sv, 1, 1)
        step(sv, dv, 0, 0, do_swait=False, do_gather=True)
        step(sv, dv, 1, 1, do_swait=True, do_gather=True)

        def trip(j, carry, sv=sv, dv=dv):
            cb = 3 * j + 2
            step(sv, dv, cb, 2, True, True)
            step(sv, dv, cb + 1, 0, True, True)
            step(sv, dv, cb + 2, 1, True, True)
            return carry

        lax.fori_loop(0, (CPS - 4) // 3, trip, 0)
        step(sv, dv, CPS - 2, (CPS - 2) % 3, True, False)
        step(sv, dv, CPS - 1, (CPS - 1) % 3, True, False)
        wait_scatter(dv, CPS - 1, (CPS - 1) % 3)
        if seg + 2 < SEG:
            load_slab(seg + 2, t)

    plsc.subcore_barrier()
    pltpu.sync_copy(acc.at[pl.ds(s * RPB, RPB)],
                    out_hbm.at[c].at[pl.ds(s * RPB, RPB)])

    @pl.when(s == 0)
    def _drain_tail():
        pltpu.sync_copy(acc.at[pl.ds(NS * RPB, TAIL)],
                        out_hbm.at[c].at[pl.ds(NS * RPB, TAIL)])


@functools.lru_cache(maxsize=None)
def _make_sc_call():
    return functools.partial(
        pl.kernel,
        out_type=jax.ShapeDtypeStruct((NC, N, EMB), jnp.float32),
        mesh=plsc.VectorSubcoreMesh(core_axis_name="c", subcore_axis_name="s",
                                    num_cores=NC, num_subcores=NS),
        scratch_types=[
            pltpu.VMEM((CPS, CH), jnp.int32),
            pltpu.VMEM((CPS, CH), jnp.int32),
            pltpu.VMEM((CPS, CH), jnp.int32),
            pltpu.VMEM((CPS, CH), jnp.int32),
            pltpu.VMEM((CH, EMB), jnp.float32),
            pltpu.VMEM((CH, EMB), jnp.float32),
            pltpu.VMEM((CH, EMB), jnp.float32),
            pltpu.VMEM_SHARED((N, EMB), jnp.float32),
            pltpu.SemaphoreType.DMA((2,)),
            pltpu.SemaphoreType.DMA((3,)),
            pltpu.SemaphoreType.DMA((3,)),
        ],
    )(_sc_body)


def kernel(x, degree, W_in, b_in, W1, b1, W2, b2, ln_s, ln_b,
           out_ln_s, out_ln_b, W_out, b_out, edge_index):
    deg = degree.reshape(N, 1)
    src = edge_index[0].reshape(NW, SEG, CPS, CH)
    dst = edge_index[1].reshape(NW, SEG, CPS, CH)
    zeros = jnp.zeros((RPB, EMB), jnp.float32)

    h = _in_call(x, W_in, b_in.reshape(1, EMB))
    for l in range(L):
        p = _msg_call(h, deg, W1[l], b1[l].reshape(1, HID))
        q = _make_sc_call()(p, src, dst, zeros)
        h = _upd_call(q[0], q[1], p, h, deg, W2[l], b2[l].reshape(1, EMB),
                      ln_s[l].reshape(1, EMB), ln_b[l].reshape(1, EMB))
    return _out_call(h, out_ln_s.reshape(1, EMB), out_ln_b.reshape(1, EMB),
                     W_out, b_out.reshape(1, OUT))


# speedup vs baseline: 3.1224x; 1.0657x over previous
import functools

import jax
import jax.numpy as jnp
from jax import lax
from jax.experimental import pallas as pl
from jax.experimental.pallas import tpu as pltpu
from jax.experimental.pallas import tpu_sc as plsc

N = 10000
E = 320000
D_IN = 128
EMB = 128
HID = 128
OUT = 64
L = 3

NC = 2
NS = 16
NW = NC * NS
EPW = E // NW
CH = 80
NCHUNK = EPW // CH
SEG = 5
CPS = NCHUNK // SEG
RPB = 624
TAIL = N - NS * RPB

ROWS_B = 1000
GRID = N // ROWS_B


def _ln_rows(t, s, b):
    mu = jnp.mean(t, axis=-1, keepdims=True)
    var = jnp.mean((t - mu) ** 2, axis=-1, keepdims=True)
    return (t - mu) * lax.rsqrt(var + 1e-5) * s + b


def _in_msg_body(x_ref, win_ref, bin_ref, deg_ref, w1_ref, b1_ref,
                 h_ref, p_ref):
    h = jax.nn.relu(
        jnp.dot(x_ref[...], win_ref[...], preferred_element_type=jnp.float32)
        + bin_ref[...]
    )
    h_ref[...] = h
    inv = lax.rsqrt(jnp.maximum(deg_ref[...] + 1.0, 1.0))
    m = jax.nn.relu(
        jnp.dot(h, w1_ref[...], preferred_element_type=jnp.float32)
        + b1_ref[...]
    )
    p_ref[...] = m * inv


def _upd_msg_body(q0_ref, q1_ref, p_ref, h_ref, deg_ref, w_ref, b_ref,
                  s_ref, lb_ref, w1_ref, b1_ref, h_ref_o, p_ref_o):
    inv = lax.rsqrt(jnp.maximum(deg_ref[...] + 1.0, 1.0))
    agg = (q0_ref[...] + q1_ref[...] + p_ref[...]) * inv
    t = jnp.dot(agg, w_ref[...], preferred_element_type=jnp.float32) + b_ref[...]
    hn = _ln_rows(t, s_ref[...], lb_ref[...]) + h_ref[...]
    h_ref_o[...] = hn
    m = jax.nn.relu(
        jnp.dot(hn, w1_ref[...], preferred_element_type=jnp.float32)
        + b1_ref[...]
    )
    p_ref_o[...] = m * inv


def _upd_out_body(q0_ref, q1_ref, p_ref, h_ref, deg_ref, w_ref, b_ref,
                  s_ref, lb_ref, os_ref, ob_ref, wo_ref, bo_ref, o_ref):
    inv = lax.rsqrt(jnp.maximum(deg_ref[...] + 1.0, 1.0))
    agg = (q0_ref[...] + q1_ref[...] + p_ref[...]) * inv
    t = jnp.dot(agg, w_ref[...], preferred_element_type=jnp.float32) + b_ref[...]
    hn = _ln_rows(t, s_ref[...], lb_ref[...]) + h_ref[...]
    t2 = _ln_rows(hn, os_ref[...], ob_ref[...])
    o_ref[...] = (
        jnp.dot(t2, wo_ref[...], preferred_element_type=jnp.float32)
        + bo_ref[...]
    )


def _row_spec(cols):
    return pl.BlockSpec((ROWS_B, cols), lambda i: (i, 0))


def _rep_spec(r, c):
    return pl.BlockSpec((r, c), lambda i: (0, 0))


_in_msg_call = pl.pallas_call(
    _in_msg_body,
    grid=(GRID,),
    in_specs=[
        _row_spec(D_IN),
        _rep_spec(D_IN, EMB),
        _rep_spec(1, EMB),
        _row_spec(1),
        _rep_spec(EMB, HID),
        _rep_spec(1, HID),
    ],
    out_specs=[_row_spec(EMB), _row_spec(HID)],
    out_shape=[
        jax.ShapeDtypeStruct((N, EMB), jnp.float32),
        jax.ShapeDtypeStruct((N, HID), jnp.float32),
    ],
)

_upd_msg_call = pl.pallas_call(
    _upd_msg_body,
    grid=(GRID,),
    in_specs=[
        _row_spec(HID),
        _row_spec(HID),
        _row_spec(HID),
        _row_spec(EMB),
        _row_spec(1),
        _rep_spec(HID, EMB),
        _rep_spec(1, EMB),
        _rep_spec(1, EMB),
        _rep_spec(1, EMB),
        _rep_spec(EMB, HID),
        _rep_spec(1, HID),
    ],
    out_specs=[_row_spec(EMB), _row_spec(HID)],
    out_shape=[
        jax.ShapeDtypeStruct((N, EMB), jnp.float32),
        jax.ShapeDtypeStruct((N, HID), jnp.float32),
    ],
)

_upd_out_call = pl.pallas_call(
    _upd_out_body,
    grid=(GRID,),
    in_specs=[
        _row_spec(HID),
        _row_spec(HID),
        _row_spec(HID),
        _row_spec(EMB),
        _row_spec(1),
        _rep_spec(HID, EMB),
        _rep_spec(1, EMB),
        _rep_spec(1, EMB),
        _rep_spec(1, EMB),
        _rep_spec(1, EMB),
        _rep_spec(1, EMB),
        _rep_spec(EMB, OUT),
        _rep_spec(1, OUT),
    ],
    out_specs=_row_spec(OUT),
    out_shape=jax.ShapeDtypeStruct((N, OUT), jnp.float32),
)


def _sc_body(p_hbm, src_hbm, dst_hbm, zeros_hbm, out_hbm,
             src_a, dst_a, src_b, dst_b, rows0, rows1, rows2, acc,
             lsem, gsem, ssem):
    rows = (rows0, rows1, rows2)
    slabs = ((src_a, dst_a), (src_b, dst_b))
    c = lax.axis_index("c")
    s = lax.axis_index("s")
    wid = c * NS + s

    def load_slab(seg, t):
        pltpu.async_copy(src_hbm.at[wid, seg], slabs[t][0], lsem.at[t])
        pltpu.async_copy(dst_hbm.at[wid, seg], slabs[t][1], lsem.at[t])

    def wait_slab(t):
        pltpu.make_async_copy(src_hbm.at[wid, 0], slabs[t][0],
                              lsem.at[t]).wait()
        pltpu.make_async_copy(dst_hbm.at[wid, 0], slabs[t][1],
                              lsem.at[t]).wait()

    load_slab(0, 0)
    load_slab(1, 1)
    pltpu.sync_copy(zeros_hbm.at[pl.ds(0, RPB)], acc.at[pl.ds(s * RPB, RPB)])

    @pl.when(s == 0)
    def _zero_tail():
        pltpu.sync_copy(zeros_hbm.at[pl.ds(0, TAIL)],
                        acc.at[pl.ds(NS * RPB, TAIL)])

    plsc.subcore_barrier()

    def issue_gather(sv, cc, b):
        pltpu.async_copy(p_hbm.at[sv.at[cc]], rows[b], gsem.at[b])

    def wait_gather(sv, cc, b):
        pltpu.make_async_copy(p_hbm.at[sv.at[cc]], rows[b],
                              gsem.at[b]).wait()

    def issue_scatter(dv, cc, b):
        pltpu.async_copy(rows[b], acc.at[dv.at[cc]], ssem.at[b], add=True)

    def wait_scatter(dv, cc, b):
        pltpu.make_async_copy(rows[b], acc.at[dv.at[cc]],
                              ssem.at[b]).wait()

    def step(sv, dv, cc, b, do_swait, do_gather):
        wait_gather(sv, cc, b)
        if do_swait:
            wait_scatter(dv, cc - 1, (b + 2) % 3)
        if do_gather:
            issue_gather(sv, cc + 2, (b + 2) % 3)
        issue_scatter(dv, cc, b)

    for seg in range(SEG):
        t = seg % 2
        sv, dv = slabs[t]
        wait_slab(t)
        issue_gather(sv, 0, 0)
        issue_gather(sv, 1, 1)
        step(sv, dv, 0, 0, do_swait=False, do_gather=True)
        step(sv, dv, 1, 1, do_swait=True, do_gather=True)

        def trip(j, carry, sv=sv, dv=dv):
            cb = 3 * j + 2
            step(sv, dv, cb, 2, True, True)
            step(sv, dv, cb + 1, 0, True, True)
            step(sv, dv, cb + 2, 1, True, True)
            return carry

        lax.fori_loop(0, (CPS - 4) // 3, trip, 0)
        step(sv, dv, CPS - 2, (CPS - 2) % 3, True, False)
        step(sv, dv, CPS - 1, (CPS - 1) % 3, True, False)
        wait_scatter(dv, CPS - 1, (CPS - 1) % 3)
        if seg + 2 < SEG:
            load_slab(seg + 2, t)

    plsc.subcore_barrier()
    pltpu.sync_copy(acc.at[pl.ds(s * RPB, RPB)],
                    out_hbm.at[c].at[pl.ds(s * RPB, RPB)])

    @pl.when(s == 0)
    def _drain_tail():
        pltpu.sync_copy(acc.at[pl.ds(NS * RPB, TAIL)],
                        out_hbm.at[c].at[pl.ds(NS * RPB, TAIL)])


@functools.lru_cache(maxsize=None)
def _make_sc_call():
    return functools.partial(
        pl.kernel,
        out_type=jax.ShapeDtypeStruct((NC, N, EMB), jnp.float32),
        mesh=plsc.VectorSubcoreMesh(core_axis_name="c", subcore_axis_name="s",
                                    num_cores=NC, num_subcores=NS),
        scratch_types=[
            pltpu.VMEM((CPS, CH), jnp.int32),
            pltpu.VMEM((CPS, CH), jnp.int32),
            pltpu.VMEM((CPS, CH), jnp.int32),
            pltpu.VMEM((CPS, CH), jnp.int32),
            pltpu.VMEM((CH, EMB), jnp.float32),
            pltpu.VMEM((CH, EMB), jnp.float32),
            pltpu.VMEM((CH, EMB), jnp.float32),
            pltpu.VMEM_SHARED((N, EMB), jnp.float32),
            pltpu.SemaphoreType.DMA((2,)),
            pltpu.SemaphoreType.DMA((3,)),
            pltpu.SemaphoreType.DMA((3,)),
        ],
    )(_sc_body)


def kernel(x, degree, W_in, b_in, W1, b1, W2, b2, ln_s, ln_b,
           out_ln_s, out_ln_b, W_out, b_out, edge_index):
    deg = degree.reshape(N, 1)
    src = edge_index[0].reshape(NW, SEG, CPS, CH)
    dst = edge_index[1].reshape(NW, SEG, CPS, CH)
    zeros = jnp.zeros((RPB, EMB), jnp.float32)

    h, p = _in_msg_call(x, W_in, b_in.reshape(1, EMB), deg,
                        W1[0], b1[0].reshape(1, HID))
    for l in range(L - 1):
        q = _make_sc_call()(p, src, dst, zeros)
        h, p = _upd_msg_call(q[0], q[1], p, h, deg, W2[l],
                             b2[l].reshape(1, EMB), ln_s[l].reshape(1, EMB),
                             ln_b[l].reshape(1, EMB), W1[l + 1],
                             b1[l + 1].reshape(1, HID))
    q = _make_sc_call()(p, src, dst, zeros)
    return _upd_out_call(q[0], q[1], p, h, deg, W2[L - 1],
                         b2[L - 1].reshape(1, EMB),
                         ln_s[L - 1].reshape(1, EMB),
                         ln_b[L - 1].reshape(1, EMB),
                         out_ln_s.reshape(1, EMB), out_ln_b.reshape(1, EMB),
                         W_out, b_out.reshape(1, OUT))
